# baseline (speedup 1.0000x reference)
import functools

import jax
import jax.numpy as jnp
from jax.experimental import pallas as pl
from jax.experimental.pallas import tpu as pltpu

TOP_K = 16
PATCH = 16
NBUF = 8


def _router_kernel(x_hbm, scores_ref, gate_ref, buf, sem, *, b, ph, pw, inv_n):
    steps = b * ph

    def start_copy(step, slot):
        bi = step // ph
        hi = step % ph
        pltpu.make_async_copy(
            x_hbm.at[bi, pl.ds(hi * PATCH, PATCH)],
            buf.at[slot],
            sem.at[slot],
        ).start()

    def wait_copy(step, slot):
        bi = step // ph
        hi = step % ph
        pltpu.make_async_copy(
            x_hbm.at[bi, pl.ds(hi * PATCH, PATCH)],
            buf.at[slot],
            sem.at[slot],
        ).wait()

    for s in range(NBUF):
        start_copy(jnp.int32(s), s)

    def loop(step, _):
        slot = jax.lax.rem(step, NBUF)
        wait_copy(step, slot)
        xb = buf[pl.ds(slot, 1), 0, 0:pw, :]
        s_vec = jnp.sum(xb[0], axis=1) * inv_n
        bi = step // ph
        hi = step % ph
        scores_ref[bi, pl.ds(hi, 1), :] = s_vec.reshape(1, pw)

        @pl.when(step + NBUF < steps)
        def _():
            start_copy(step + NBUF, slot)

        return 0

    jax.lax.fori_loop(0, steps, loop, 0, unroll=False)

    for bi in range(b):
        scores = scores_ref[bi]

        def body(_, carry):
            vals, cnt, thresh, done = carry
            m = jnp.max(vals)
            n = jnp.sum((vals == m).astype(jnp.int32))
            reached = cnt + n >= TOP_K
            take = jnp.logical_and(done == 0, reached)
            thresh = jnp.where(take, m, thresh)
            vals = jnp.where(
                jnp.logical_and(vals == m, done == 0), -jnp.inf, vals
            )
            cnt = jnp.where(done == 0, cnt + n, cnt)
            done = jnp.where(reached, jnp.int32(1), done)
            return vals, cnt, thresh, done

        init = (scores, jnp.int32(0), jnp.float32(-jnp.inf), jnp.int32(0))
        _, _, thresh, _ = jax.lax.fori_loop(0, TOP_K, body, init)
        gate_ref[bi] = (scores >= thresh).astype(jnp.float32)


def kernel(x):
    b, h, w, c = x.shape
    ph = h // PATCH
    pw = w // PATCH
    inv_n = 1.0 / float(PATCH * PATCH * c)

    kfn = functools.partial(_router_kernel, b=b, ph=ph, pw=pw, inv_n=inv_n)
    scores, gate = pl.pallas_call(
        kfn,
        in_specs=[pl.BlockSpec(memory_space=pltpu.HBM)],
        out_specs=[
            pl.BlockSpec(memory_space=pltpu.VMEM),
            pl.BlockSpec(memory_space=pltpu.VMEM),
        ],
        out_shape=[
            jax.ShapeDtypeStruct((b, ph, pw), jnp.float32),
            jax.ShapeDtypeStruct((b, ph, pw), jnp.float32),
        ],
        scratch_shapes=[
            pltpu.VMEM((NBUF, PATCH, w, c), jnp.float32),
            pltpu.SemaphoreType.DMA((NBUF,)),
        ],
    )(x)
    return scores.reshape(b, ph * pw), gate.reshape(b, ph * pw)


# speedup vs baseline: 3.0335x; 3.0335x over previous
import functools

import jax
import jax.numpy as jnp
from jax.experimental import pallas as pl
from jax.experimental.pallas import tpu as pltpu

TOP_K = 16
PATCH = 16


def _router_kernel(x_ref, scores_ref, gate_ref, *, ph, pw, inv_n):
    i = pl.program_id(1)

    xb = x_ref[0]
    t = jnp.sum(jnp.abs(xb), axis=0)
    u = jnp.sum(t, axis=0, keepdims=True)
    w = u.shape[1]
    seg = (
        jax.lax.broadcasted_iota(jnp.int32, (w, pw), 0) // PATCH
        == jax.lax.broadcasted_iota(jnp.int32, (w, pw), 1)
    ).astype(jnp.float32)
    s = jnp.dot(
        u, seg,
        preferred_element_type=jnp.float32,
        precision=jax.lax.Precision.HIGHEST,
    )[0] * inv_n
    scores_ref[0, i, :] = s

    @pl.when(i == ph - 1)
    def _():
        scores = scores_ref[0]

        def body(_, carry):
            vals, cnt, thresh, done = carry
            m = jnp.max(vals)
            n = jnp.sum((vals == m).astype(jnp.int32))
            reached = cnt + n >= TOP_K
            take = jnp.logical_and(done == 0, reached)
            thresh = jnp.where(take, m, thresh)
            vals = jnp.where(
                jnp.logical_and(vals == m, done == 0), -jnp.inf, vals
            )
            cnt = jnp.where(done == 0, cnt + n, cnt)
            done = jnp.where(reached, jnp.int32(1), done)
            return vals, cnt, thresh, done

        init = (scores, jnp.int32(0), jnp.float32(-jnp.inf), jnp.int32(0))
        _, _, thresh, _ = jax.lax.fori_loop(0, TOP_K, body, init)
        gate_ref[0] = (scores >= thresh).astype(jnp.float32)


def kernel(x):
    b, h, w, c = x.shape
    ph = h // PATCH
    pw = w // PATCH
    inv_n = 1.0 / float(PATCH * PATCH * c)
    xt = jnp.swapaxes(x, 2, 3)

    kfn = functools.partial(_router_kernel, ph=ph, pw=pw, inv_n=inv_n)
    scores, gate = pl.pallas_call(
        kfn,
        grid=(b, ph),
        in_specs=[
            pl.BlockSpec(
                (1, PATCH, c, w),
                lambda bi, i: (bi, i, 0, 0),
            )
        ],
        out_specs=[
            pl.BlockSpec((1, ph, pw), lambda bi, i: (bi, 0, 0)),
            pl.BlockSpec((1, ph, pw), lambda bi, i: (bi, 0, 0)),
        ],
        out_shape=[
            jax.ShapeDtypeStruct((b, ph, pw), jnp.float32),
            jax.ShapeDtypeStruct((b, ph, pw), jnp.float32),
        ],
        compiler_params=pltpu.CompilerParams(
            dimension_semantics=("arbitrary", "arbitrary"),
        ),
    )(xt)
    return scores.reshape(b, ph * pw), gate.reshape(b, ph * pw)
